# baseline (speedup 1.0000x reference)
import functools

import jax
import jax.numpy as jnp
from jax import lax
from jax.experimental import pallas as pl
from jax.experimental.pallas import tpu as pltpu
from jax.experimental.pallas import tpu_sc as plsc

_F = 26
_B = 16384
_D = 32
_N = _B * _F
_VOCAB_PER_FIELD = 100000

_NC = 2
_NS = 16
_NW = _NC * _NS
_PER_W = _N // _NW
_G = 128
_CHG = 8
_CH = _CHG * _G
_NCH = _PER_W // _CH


def _sc_body(idx_hbm, table_hbm, out_hbm, idx_v, rows_v, sem):
    wid = lax.axis_index("s") * _NC + lax.axis_index("c")
    lane = lax.iota(jnp.int32, 16)

    def do_chunk(c, _):
        row0 = pl.multiple_of(wid * _PER_W + c * _CH, _CH)
        pltpu.sync_copy(
            idx_hbm.at[pl.ds(pl.multiple_of(row0 // _G, _CHG), _CHG)], idx_v
        )

        def fixup(i, _):
            j = i // 8
            s = pl.ds((i % 8) * 16, 16)
            fld = (row0 + i * 16 + lane) % _F
            idx_v[j, s] = idx_v[j, s] + fld * _VOCAB_PER_FIELD
            return 0

        lax.fori_loop(0, _CH // 16, fixup, 0)

        copies = [
            pltpu.async_copy(
                table_hbm.at[idx_v.at[j]],
                rows_v.at[pl.ds(j * _G, _G)],
                sem,
            )
            for j in range(_CHG)
        ]
        for cp in copies:
            cp.wait()

        pltpu.sync_copy(rows_v, out_hbm.at[pl.ds(row0, _CH)])
        return 0

    lax.fori_loop(0, _NCH, do_chunk, 0)


@jax.jit
def _sc_gather(idx2d, table):
    mesh = plsc.VectorSubcoreMesh(core_axis_name="c", subcore_axis_name="s")
    run = functools.partial(
        pl.kernel,
        mesh=mesh,
        out_type=jax.ShapeDtypeStruct((_N, _D), jnp.float32),
        scratch_types=[
            pltpu.VMEM((_CHG, _G), jnp.int32),
            pltpu.VMEM((_CH, _D), jnp.float32),
            pltpu.SemaphoreType.DMA,
        ],
        compiler_params=pltpu.CompilerParams(use_tc_tiling_on_sc=False),
    )(_sc_body)
    return run(idx2d, table)


def kernel(input_x, table):
    idx2d = input_x.reshape(_N // _G, _G)
    out = _sc_gather(idx2d, table)
    return out.reshape(_B, _F, _D)


# speedup vs baseline: 1.0105x; 1.0105x over previous
import functools

import jax
import jax.numpy as jnp
from jax import lax
from jax.experimental import pallas as pl
from jax.experimental.pallas import tpu as pltpu
from jax.experimental.pallas import tpu_sc as plsc

_F = 26
_B = 16384
_D = 32
_N = _B * _F
_VOCAB_PER_FIELD = 100000

_NC = 2
_NS = 16
_NW = _NC * _NS
_PER_W = _N // _NW
_G = 128
_CHG = 8
_CH = _CHG * _G
_NCH = _PER_W // _CH


def _sc_body(idx_hbm, table_hbm, out_hbm, idx_v, rows_v,
             sem_idx, sem_g, sem_out):
    wid = lax.axis_index("s") * _NC + lax.axis_index("c")
    lane = lax.iota(jnp.int32, 16)
    base = wid * _PER_W

    def row0_of(c):
        return pl.multiple_of(base + c * _CH, _CH)

    def idx_load(c):
        p = c % 2
        r0 = pl.multiple_of(row0_of(c) // _G, _CHG)
        return pltpu.async_copy(
            idx_hbm.at[pl.ds(r0, _CHG)], idx_v.at[p], sem_idx.at[p]
        )

    def fixup(c):
        p = c % 2
        row0 = row0_of(c)

        def body(i, _):
            j = i // 8
            s = pl.ds((i % 8) * 16, 16)
            fld = (row0 + i * 16 + lane) % _F
            idx_v[p, j, s] = idx_v[p, j, s] + fld * _VOCAB_PER_FIELD
            return 0

        lax.fori_loop(0, _CH // 16, body, 0)

    def fire_gathers(c):
        p = c % 2
        return [
            pltpu.async_copy(
                table_hbm.at[idx_v.at[p, j]],
                rows_v.at[p, pl.ds(j * _G, _G)],
                sem_g.at[p],
            )
            for j in range(_CHG)
        ]

    def out_store(c):
        p = c % 2
        return pltpu.async_copy(
            rows_v.at[p], out_hbm.at[pl.ds(row0_of(c), _CH)], sem_out.at[p]
        )

    h_idx = [None] * _NCH
    h_g = [None] * _NCH
    h_out = [None] * _NCH

    h_idx[0] = idx_load(0)
    h_idx[1] = idx_load(1)
    h_idx[0].wait()
    fixup(0)
    h_g[0] = fire_gathers(0)

    for c in range(_NCH):
        if c + 1 < _NCH:
            h_idx[c + 1].wait()
            fixup(c + 1)
            if c >= 1:
                h_out[c - 1].wait()
            h_g[c + 1] = fire_gathers(c + 1)
        for h in h_g[c]:
            h.wait()
        if c + 2 < _NCH:
            h_idx[c + 2] = idx_load(c + 2)
        h_out[c] = out_store(c)

    h_out[_NCH - 2].wait()
    h_out[_NCH - 1].wait()


@jax.jit
def _sc_gather(idx2d, table):
    mesh = plsc.VectorSubcoreMesh(core_axis_name="c", subcore_axis_name="s")
    run = functools.partial(
        pl.kernel,
        mesh=mesh,
        out_type=jax.ShapeDtypeStruct((_N, _D), jnp.float32),
        scratch_types=[
            pltpu.VMEM((2, _CHG, _G), jnp.int32),
            pltpu.VMEM((2, _CH, _D), jnp.float32),
            pltpu.SemaphoreType.DMA((2,)),
            pltpu.SemaphoreType.DMA((2,)),
            pltpu.SemaphoreType.DMA((2,)),
        ],
        compiler_params=pltpu.CompilerParams(use_tc_tiling_on_sc=False),
    )(_sc_body)
    return run(idx2d, table)


def kernel(input_x, table):
    idx2d = input_x.reshape(_N // _G, _G)
    out = _sc_gather(idx2d, table)
    return out.reshape(_B, _F, _D)
